# baseline (device time: 148337 ns/iter reference)
import jax
import jax.numpy as jnp
from jax import lax
from jax.experimental import pallas as pl
from jax.experimental.pallas import tpu as pltpu

N_DEV = 8
N_TOK = 2048
D_MODEL = 1024
N_EXPERTS = 64
E_LOCAL = N_EXPERTS // N_DEV
CHUNK = N_TOK // N_DEV


def kernel(x, router_W, route_idx, expert_W):
    r = lax.axis_index("i")

    scores = x @ router_W
    probs = jax.nn.softmax(scores, axis=-1)
    g = jnp.take_along_axis(probs, route_idx, axis=1)
    gsum = g.sum(axis=1, keepdims=True)
    e_ids = jnp.arange(N_EXPERTS, dtype=jnp.int32)[None, :]
    w_full = (route_idx[:, 0:1] == e_ids) * (g[:, 0:1] / gsum) + (
        route_idx[:, 1:2] == e_ids
    ) * (g[:, 1:2] / gsum)
    gates = lax.dynamic_slice(w_full, (0, r * E_LOCAL), (N_TOK, E_LOCAL))

    xb = x.astype(jnp.bfloat16)
    wb = expert_W.astype(jnp.bfloat16)

    def body(x_ref, g_ref, w_ref, out_ref, comm_ref, send_sems, recv_sems):
        my = lax.axis_index("i")
        left = lax.rem(my + N_DEV - 1, N_DEV)
        right = lax.rem(my + 1, N_DEV)

        barrier = pltpu.get_barrier_semaphore()
        for nbr in (left, right):
            pl.semaphore_signal(
                barrier, inc=1, device_id=(nbr,),
                device_id_type=pl.DeviceIdType.MESH,
            )
        pl.semaphore_wait(barrier, 2)

        def partial_chunk(c):
            rows = pl.ds(c * CHUNK, CHUNK)
            xc = x_ref[rows, :]
            gc = g_ref[rows, :]
            acc = jnp.zeros((CHUNK, D_MODEL), jnp.float32)
            for j in range(E_LOCAL):
                y = jnp.dot(xc, w_ref[j], preferred_element_type=jnp.float32)
                acc = acc + gc[:, j : j + 1] * y
            return acc

        comm_ref[0, :, :] = partial_chunk(lax.rem(my + N_DEV - 1, N_DEV))

        for s in range(N_DEV - 1):
            rdma = pltpu.make_async_remote_copy(
                src_ref=comm_ref.at[s],
                dst_ref=comm_ref.at[s + 1],
                send_sem=send_sems.at[s],
                recv_sem=recv_sems.at[s],
                device_id=(right,),
                device_id_type=pl.DeviceIdType.MESH,
            )
            rdma.start()
            acc = partial_chunk(lax.rem(my + 2 * N_DEV - s - 2, N_DEV))
            rdma.wait()
            if s < N_DEV - 2:
                comm_ref[s + 1, :, :] = comm_ref[s + 1, :, :] + acc
            else:
                out_ref[:, :] = comm_ref[s + 1, :, :] + acc

    return pl.pallas_call(
        body,
        out_shape=jax.ShapeDtypeStruct((CHUNK, D_MODEL), jnp.float32),
        in_specs=[
            pl.BlockSpec(memory_space=pltpu.VMEM),
            pl.BlockSpec(memory_space=pltpu.VMEM),
            pl.BlockSpec(memory_space=pltpu.VMEM),
        ],
        out_specs=pl.BlockSpec(memory_space=pltpu.VMEM),
        scratch_shapes=[
            pltpu.VMEM((N_DEV, CHUNK, D_MODEL), jnp.float32),
            pltpu.SemaphoreType.DMA((N_DEV - 1,)),
            pltpu.SemaphoreType.DMA((N_DEV - 1,)),
        ],
        compiler_params=pltpu.CompilerParams(collective_id=0),
    )(xb, gates, wb)


# device time: 88478 ns/iter; 1.6765x vs baseline; 1.6765x over previous
import jax
import jax.numpy as jnp
from jax import lax
from jax.experimental import pallas as pl
from jax.experimental.pallas import tpu as pltpu

N_DEV = 8
N_TOK = 2048
D_MODEL = 1024
N_EXPERTS = 64
E_LOCAL = N_EXPERTS // N_DEV
CHUNK = N_TOK // N_DEV


def kernel(x, router_W, route_idx, expert_W):
    def body(
        x_ref,
        rw_ref,
        idx_ref,
        w_hbm,
        out_ref,
        xb_ref,
        gates_ref,
        wb_ref,
        w_stage,
        comm_ref,
        w_sems,
        send_sems,
        recv_sems,
    ):
        my = lax.axis_index("i")
        left = lax.rem(my + N_DEV - 1, N_DEV)
        right = lax.rem(my + 1, N_DEV)

        barrier = pltpu.get_barrier_semaphore()
        for nbr in (left, right):
            pl.semaphore_signal(
                barrier, inc=1, device_id=(nbr,),
                device_id_type=pl.DeviceIdType.MESH,
            )

        def wcopy(j, slot):
            return pltpu.make_async_copy(
                w_hbm.at[j], w_stage.at[slot], w_sems.at[slot]
            )

        wcopy(0, 0).start()
        wcopy(1, 1).start()

        xb_ref[:, :] = x_ref[:, :].astype(jnp.bfloat16)
        scores = jnp.dot(
            x_ref[:, :], rw_ref[:, :], preferred_element_type=jnp.float32
        )
        s_max = jnp.max(scores, axis=1, keepdims=True)
        e = jnp.exp(scores - s_max)
        probs = e / jnp.sum(e, axis=1, keepdims=True)
        idx0 = idx_ref[:, 0:1]
        idx1 = idx_ref[:, 1:2]
        lanes64 = lax.broadcasted_iota(jnp.int32, (N_TOK, N_EXPERTS), 1)
        g0 = jnp.sum(
            jnp.where(lanes64 == idx0, probs, 0.0), axis=1, keepdims=True
        )
        g1 = jnp.sum(
            jnp.where(lanes64 == idx1, probs, 0.0), axis=1, keepdims=True
        )
        gsum = g0 + g1
        local_ids = my * E_LOCAL + lax.broadcasted_iota(
            jnp.int32, (N_TOK, E_LOCAL), 1
        )
        gates_ref[:, :] = (
            jnp.where(local_ids == idx0, g0 / gsum, 0.0)
            + jnp.where(local_ids == idx1, g1 / gsum, 0.0)
        )

        for j in range(E_LOCAL):
            wcopy(j, j % 2).wait()
            wb_ref[j, :, :] = w_stage[j % 2, :, :].astype(jnp.bfloat16)
            if j + 2 < E_LOCAL:
                wcopy(j + 2, j % 2).start()

        def partial_chunk(c):
            rows = pl.ds(c * CHUNK, CHUNK)
            xc = xb_ref[rows, :]
            gc = gates_ref[rows, :]
            acc = jnp.zeros((CHUNK, D_MODEL), jnp.float32)
            for j in range(E_LOCAL):
                y = jnp.dot(xc, wb_ref[j], preferred_element_type=jnp.float32)
                acc = acc + gc[:, j : j + 1] * y
            return acc

        pl.semaphore_wait(barrier, 2)

        comm_ref[0, :, :] = partial_chunk(
            lax.rem(my + N_DEV - 1, N_DEV)
        ).astype(jnp.bfloat16)

        for s in range(N_DEV - 1):
            rdma = pltpu.make_async_remote_copy(
                src_ref=comm_ref.at[s],
                dst_ref=comm_ref.at[s + 1],
                send_sem=send_sems.at[s],
                recv_sem=recv_sems.at[s],
                device_id=(right,),
                device_id_type=pl.DeviceIdType.MESH,
            )
            rdma.start()
            acc = partial_chunk(lax.rem(my + 2 * N_DEV - s - 2, N_DEV))
            rdma.wait()
            if s < N_DEV - 2:
                comm_ref[s + 1, :, :] = (
                    comm_ref[s + 1, :, :].astype(jnp.float32) + acc
                ).astype(jnp.bfloat16)
            else:
                out_ref[:, :] = comm_ref[s + 1, :, :].astype(jnp.float32) + acc

    return pl.pallas_call(
        body,
        out_shape=jax.ShapeDtypeStruct((CHUNK, D_MODEL), jnp.float32),
        in_specs=[
            pl.BlockSpec(memory_space=pltpu.VMEM),
            pl.BlockSpec(memory_space=pltpu.VMEM),
            pl.BlockSpec(memory_space=pltpu.VMEM),
            pl.BlockSpec(memory_space=pltpu.MemorySpace.HBM),
        ],
        out_specs=pl.BlockSpec(memory_space=pltpu.VMEM),
        scratch_shapes=[
            pltpu.VMEM((N_TOK, D_MODEL), jnp.bfloat16),
            pltpu.VMEM((N_TOK, E_LOCAL), jnp.float32),
            pltpu.VMEM((E_LOCAL, D_MODEL, D_MODEL), jnp.bfloat16),
            pltpu.VMEM((2, D_MODEL, D_MODEL), jnp.float32),
            pltpu.VMEM((N_DEV, CHUNK, D_MODEL), jnp.bfloat16),
            pltpu.SemaphoreType.DMA((2,)),
            pltpu.SemaphoreType.DMA((N_DEV - 1,)),
            pltpu.SemaphoreType.DMA((N_DEV - 1,)),
        ],
        compiler_params=pltpu.CompilerParams(
            collective_id=0, vmem_limit_bytes=64 * 1024 * 1024
        ),
    )(x, router_W, route_idx, expert_W)


# device time: 68856 ns/iter; 2.1543x vs baseline; 1.2850x over previous
import jax
import jax.numpy as jnp
from jax import lax
from jax.experimental import pallas as pl
from jax.experimental.pallas import tpu as pltpu

N_DEV = 8
N_TOK = 2048
D_MODEL = 1024
N_EXPERTS = 64
E_LOCAL = N_EXPERTS // N_DEV
CHUNK = N_TOK // N_DEV


def kernel(x, router_W, route_idx, expert_W):
    def body(
        x_ref,
        rw_ref,
        idx_ref,
        w_hbm,
        out_ref,
        xb_ref,
        gates_ref,
        wb_ref,
        w_stage,
        part_ref,
        recv_ref,
        w_sems,
        send_sems,
        recv_sems,
    ):
        my = lax.axis_index("i")

        barrier = pltpu.get_barrier_semaphore()
        for k in range(1, N_DEV):
            pl.semaphore_signal(
                barrier, inc=1, device_id=(lax.rem(my + k, N_DEV),),
                device_id_type=pl.DeviceIdType.MESH,
            )

        def wcopy(j, slot):
            return pltpu.make_async_copy(
                w_hbm.at[j], w_stage.at[slot], w_sems.at[slot]
            )

        wcopy(0, 0).start()
        wcopy(1, 1).start()

        xb_ref[:, :] = x_ref[:, :].astype(jnp.bfloat16)
        scores = jnp.dot(
            x_ref[:, :], rw_ref[:, :], preferred_element_type=jnp.float32
        )
        s_max = jnp.max(scores, axis=1, keepdims=True)
        e = jnp.exp(scores - s_max)
        probs = e / jnp.sum(e, axis=1, keepdims=True)
        idx0 = idx_ref[:, 0:1]
        idx1 = idx_ref[:, 1:2]
        lanes64 = lax.broadcasted_iota(jnp.int32, (N_TOK, N_EXPERTS), 1)
        g0 = jnp.sum(
            jnp.where(lanes64 == idx0, probs, 0.0), axis=1, keepdims=True
        )
        g1 = jnp.sum(
            jnp.where(lanes64 == idx1, probs, 0.0), axis=1, keepdims=True
        )
        gsum = g0 + g1
        local_ids = my * E_LOCAL + lax.broadcasted_iota(
            jnp.int32, (N_TOK, E_LOCAL), 1
        )
        gates_ref[:, :] = (
            jnp.where(local_ids == idx0, g0 / gsum, 0.0)
            + jnp.where(local_ids == idx1, g1 / gsum, 0.0)
        )

        def expert_mm(c, j):
            rows = pl.ds(c * CHUNK, CHUNK)
            y = jnp.dot(
                xb_ref[rows, :], wb_ref[j], preferred_element_type=jnp.float32
            )
            return gates_ref[rows, j : j + 1] * y

        c_first = lax.rem(my + 1, N_DEV)
        acc1 = jnp.zeros((CHUNK, D_MODEL), jnp.float32)
        for j in range(E_LOCAL):
            wcopy(j, j % 2).wait()
            wb_ref[j, :, :] = w_stage[j % 2, :, :].astype(jnp.bfloat16)
            if j + 2 < E_LOCAL:
                wcopy(j + 2, j % 2).start()
            acc1 = acc1 + expert_mm(c_first, j)

        def partial_chunk(c):
            acc = jnp.zeros((CHUNK, D_MODEL), jnp.float32)
            for j in range(E_LOCAL):
                acc = acc + expert_mm(c, j)
            return acc

        pl.semaphore_wait(barrier, N_DEV - 1)

        sends = []
        for k in range(1, N_DEV):
            c = lax.rem(my + k, N_DEV)
            acc = acc1 if k == 1 else partial_chunk(c)
            part_ref[k - 1, :, :] = acc.astype(jnp.bfloat16)
            rdma = pltpu.make_async_remote_copy(
                src_ref=part_ref.at[k - 1],
                dst_ref=recv_ref.at[N_DEV - 1 - k],
                send_sem=send_sems.at[k - 1],
                recv_sem=recv_sems.at[N_DEV - 1 - k],
                device_id=(c,),
                device_id_type=pl.DeviceIdType.MESH,
            )
            rdma.start()
            sends.append(rdma)

        acc = partial_chunk(my)
        for slot in range(N_DEV - 2, -1, -1):
            recv = pltpu.make_async_remote_copy(
                src_ref=part_ref.at[0],
                dst_ref=recv_ref.at[slot],
                send_sem=send_sems.at[0],
                recv_sem=recv_sems.at[slot],
                device_id=(my,),
                device_id_type=pl.DeviceIdType.MESH,
            )
            recv.wait_recv()
            acc = acc + recv_ref[slot, :, :].astype(jnp.float32)
        out_ref[:, :] = acc

        for rdma in sends:
            rdma.wait_send()

    return pl.pallas_call(
        body,
        out_shape=jax.ShapeDtypeStruct((CHUNK, D_MODEL), jnp.float32),
        in_specs=[
            pl.BlockSpec(memory_space=pltpu.VMEM),
            pl.BlockSpec(memory_space=pltpu.VMEM),
            pl.BlockSpec(memory_space=pltpu.VMEM),
            pl.BlockSpec(memory_space=pltpu.MemorySpace.HBM),
        ],
        out_specs=pl.BlockSpec(memory_space=pltpu.VMEM),
        scratch_shapes=[
            pltpu.VMEM((N_TOK, D_MODEL), jnp.bfloat16),
            pltpu.VMEM((N_TOK, E_LOCAL), jnp.float32),
            pltpu.VMEM((E_LOCAL, D_MODEL, D_MODEL), jnp.bfloat16),
            pltpu.VMEM((2, D_MODEL, D_MODEL), jnp.float32),
            pltpu.VMEM((N_DEV - 1, CHUNK, D_MODEL), jnp.bfloat16),
            pltpu.VMEM((N_DEV - 1, CHUNK, D_MODEL), jnp.bfloat16),
            pltpu.SemaphoreType.DMA((2,)),
            pltpu.SemaphoreType.DMA((N_DEV - 1,)),
            pltpu.SemaphoreType.DMA((N_DEV - 1,)),
        ],
        compiler_params=pltpu.CompilerParams(
            collective_id=0, vmem_limit_bytes=64 * 1024 * 1024
        ),
    )(x, router_W, route_idx, expert_W)
